# baseline (device time: 59740 ns/iter reference)
import jax
import jax.numpy as jnp
from jax import lax
from jax.experimental import pallas as pl
from jax.experimental.pallas import tpu as pltpu

N_DEV = 4


def kernel(x, router_W, route_idx, expert_W, shared_W):
    n_tok, d_model = x.shape
    n_experts = router_W.shape[1]
    n_loc, _, d_ff = expert_W.shape
    chunk = n_tok // N_DEV

    def body(x_ref, rw_ref, ridx_ref, ew_ref, sw_ref, out_ref,
             probs_ref, ew_bf_ref, comm_ref, send_sems, recv_sems):
        my = lax.axis_index("i")
        left = lax.rem(my + N_DEV - 1, N_DEV)
        right = lax.rem(my + 1, N_DEV)

        scores = jnp.dot(
            x_ref[:].astype(jnp.bfloat16),
            rw_ref[:].astype(jnp.bfloat16),
            preferred_element_type=jnp.float32,
        )
        m = jnp.max(scores, axis=1, keepdims=True)
        p = jnp.exp(scores - m)
        probs_ref[:] = p / jnp.sum(p, axis=1, keepdims=True)

        ew_bf_ref[:] = ew_ref[:].astype(jnp.bfloat16)

        def partial_chunk(c):
            row0 = c * chunk
            xc = x_ref[pl.ds(row0, chunk), :]
            ridx = ridx_ref[pl.ds(row0, chunk), :]
            pc = probs_ref[pl.ds(row0, chunk), :]
            col = lax.broadcasted_iota(jnp.int32, (chunk, n_experts), 1)
            acc = jnp.zeros((chunk, d_ff), jnp.float32)
            for le in range(n_loc):
                e = my * n_loc + le
                prob_e = jnp.sum(
                    jnp.where(col == e, pc, 0.0), axis=1, keepdims=True
                )
                gate = jnp.where(ridx == e, prob_e, 0.0)
                xs = (xc * gate).astype(jnp.bfloat16)
                acc = acc + jnp.dot(
                    xs, ew_bf_ref[le], preferred_element_type=jnp.float32
                )
            return acc

        c0 = lax.rem(my + N_DEV - 1, N_DEV)
        comm_ref[0] = partial_chunk(c0).astype(jnp.bfloat16)

        barrier_sem = pltpu.get_barrier_semaphore()
        for nbr in (left, right):
            pl.semaphore_signal(
                barrier_sem, inc=1,
                device_id=(nbr,), device_id_type=pl.DeviceIdType.MESH,
            )
        pl.semaphore_wait(barrier_sem, 2)

        for s in range(N_DEV - 1):
            rdma = pltpu.make_async_remote_copy(
                src_ref=comm_ref.at[s],
                dst_ref=comm_ref.at[s + 1],
                send_sem=send_sems.at[s],
                recv_sem=recv_sems.at[s],
                device_id=(right,),
                device_id_type=pl.DeviceIdType.MESH,
            )
            rdma.start()
            c = lax.rem(my + 2 * N_DEV - 2 - s, N_DEV)
            acc = partial_chunk(c)
            if s == N_DEV - 2:
                xm = x_ref[pl.ds(my * chunk, chunk), :].astype(jnp.bfloat16)
                acc = acc + jnp.dot(
                    xm, sw_ref[:].astype(jnp.bfloat16),
                    preferred_element_type=jnp.float32,
                )
            rdma.wait_recv()
            rdma.wait_send()
            if s < N_DEV - 2:
                comm_ref[s + 1] = (
                    comm_ref[s + 1].astype(jnp.float32) + acc
                ).astype(jnp.bfloat16)
            else:
                out_ref[:] = comm_ref[s + 1].astype(jnp.float32) + acc

    return pl.pallas_call(
        body,
        out_shape=jax.ShapeDtypeStruct((chunk, d_ff), jnp.float32),
        in_specs=[pl.BlockSpec(memory_space=pltpu.VMEM)] * 5,
        out_specs=pl.BlockSpec(memory_space=pltpu.VMEM),
        scratch_shapes=[
            pltpu.VMEM((n_tok, n_experts), jnp.float32),
            pltpu.VMEM((n_loc, d_model, d_ff), jnp.bfloat16),
            pltpu.VMEM((N_DEV, chunk, d_ff), jnp.bfloat16),
            pltpu.SemaphoreType.DMA((N_DEV - 1,)),
            pltpu.SemaphoreType.DMA((N_DEV - 1,)),
        ],
        compiler_params=pltpu.CompilerParams(collective_id=0),
    )(x, router_W, route_idx, expert_W, shared_W)


# device time: 49819 ns/iter; 1.1991x vs baseline; 1.1991x over previous
import jax
import jax.numpy as jnp
from jax import lax
from jax.experimental import pallas as pl
from jax.experimental.pallas import tpu as pltpu

N_DEV = 4


def kernel(x, router_W, route_idx, expert_W, shared_W):
    n_tok, d_model = x.shape
    n_experts = router_W.shape[1]
    n_loc, _, d_ff = expert_W.shape
    chunk = n_tok // N_DEV

    def body(x_ref, rw_ref, ridx_ref, ew_ref, sw_ref, out_ref,
             probs_ref, ew_bf_ref, send_ref, recv_ref, send_sems, recv_sems):
        my = lax.axis_index("i")

        scores = jnp.dot(
            x_ref[:].astype(jnp.bfloat16),
            rw_ref[:].astype(jnp.bfloat16),
            preferred_element_type=jnp.float32,
        )
        m = jnp.max(scores, axis=1, keepdims=True)
        p = jnp.exp(scores - m)
        probs_ref[:] = p / jnp.sum(p, axis=1, keepdims=True)

        ew_bf_ref[:] = ew_ref[:].astype(jnp.bfloat16)

        def partial_chunk(c):
            row0 = c * chunk
            xc = x_ref[pl.ds(row0, chunk), :]
            ridx = ridx_ref[pl.ds(row0, chunk), :]
            pc = probs_ref[pl.ds(row0, chunk), :]
            col = lax.broadcasted_iota(jnp.int32, (chunk, n_experts), 1)
            acc = jnp.zeros((chunk, d_ff), jnp.float32)
            for le in range(n_loc):
                e = my * n_loc + le
                prob_e = jnp.sum(
                    jnp.where(col == e, pc, 0.0), axis=1, keepdims=True
                )
                gate = jnp.where(ridx == e, prob_e, 0.0)
                xs = (xc * gate).astype(jnp.bfloat16)
                acc = acc + jnp.dot(
                    xs, ew_bf_ref[le], preferred_element_type=jnp.float32
                )
            return acc

        barrier_sem = pltpu.get_barrier_semaphore()
        for t in range(1, N_DEV):
            pl.semaphore_signal(
                barrier_sem, inc=1,
                device_id=(lax.rem(my + t, N_DEV),),
                device_id_type=pl.DeviceIdType.MESH,
            )
        pl.semaphore_wait(barrier_sem, N_DEV - 1)

        rdmas = []
        for t in range(1, N_DEV):
            dst = lax.rem(my + t, N_DEV)
            send_ref[t - 1] = partial_chunk(dst).astype(jnp.bfloat16)
            rdma = pltpu.make_async_remote_copy(
                src_ref=send_ref.at[t - 1],
                dst_ref=recv_ref.at[t - 1],
                send_sem=send_sems.at[t - 1],
                recv_sem=recv_sems.at[t - 1],
                device_id=(dst,),
                device_id_type=pl.DeviceIdType.MESH,
            )
            rdma.start()
            rdmas.append(rdma)

        own = partial_chunk(my)
        xm = x_ref[pl.ds(my * chunk, chunk), :].astype(jnp.bfloat16)
        own = own + jnp.dot(
            xm, sw_ref[:].astype(jnp.bfloat16),
            preferred_element_type=jnp.float32,
        )

        for rdma in rdmas:
            rdma.wait_recv()
        out_ref[:] = (
            own
            + recv_ref[0].astype(jnp.float32)
            + recv_ref[1].astype(jnp.float32)
            + recv_ref[2].astype(jnp.float32)
        )
        for rdma in rdmas:
            rdma.wait_send()

    return pl.pallas_call(
        body,
        out_shape=jax.ShapeDtypeStruct((chunk, d_ff), jnp.float32),
        in_specs=[pl.BlockSpec(memory_space=pltpu.VMEM)] * 5,
        out_specs=pl.BlockSpec(memory_space=pltpu.VMEM),
        scratch_shapes=[
            pltpu.VMEM((n_tok, n_experts), jnp.float32),
            pltpu.VMEM((n_loc, d_model, d_ff), jnp.bfloat16),
            pltpu.VMEM((N_DEV - 1, chunk, d_ff), jnp.bfloat16),
            pltpu.VMEM((N_DEV - 1, chunk, d_ff), jnp.bfloat16),
            pltpu.SemaphoreType.DMA((N_DEV - 1,)),
            pltpu.SemaphoreType.DMA((N_DEV - 1,)),
        ],
        compiler_params=pltpu.CompilerParams(collective_id=0),
    )(x, router_W, route_idx, expert_W, shared_W)


# device time: 44522 ns/iter; 1.3418x vs baseline; 1.1190x over previous
import jax
import jax.numpy as jnp
from jax import lax
from jax.experimental import pallas as pl
from jax.experimental.pallas import tpu as pltpu

N_DEV = 4
WIRE_DTYPE = "int8"


def kernel(x, router_W, route_idx, expert_W, shared_W):
    n_tok, d_model = x.shape
    n_experts = router_W.shape[1]
    n_loc, _, d_ff = expert_W.shape
    chunk = n_tok // N_DEV
    int8_wire = WIRE_DTYPE == "int8"
    wire_dt = jnp.int8 if int8_wire else jnp.bfloat16

    def body(x_ref, rw_ref, ridx_ref, ew_ref, sw_ref, out_ref,
             probs_ref, ew_bf_ref, send_ref, recv_ref,
             sscale_ref, rscale_ref,
             send_sems, recv_sems, s2_sems, r2_sems):
        my = lax.axis_index("i")

        scores = jnp.dot(
            x_ref[:].astype(jnp.bfloat16),
            rw_ref[:].astype(jnp.bfloat16),
            preferred_element_type=jnp.float32,
        )
        m = jnp.max(scores, axis=1, keepdims=True)
        p = jnp.exp(scores - m)
        probs_ref[:] = p / jnp.sum(p, axis=1, keepdims=True)

        ew_bf_ref[:] = ew_ref[:].astype(jnp.bfloat16)

        def partial_chunk(c):
            row0 = c * chunk
            xc = x_ref[pl.ds(row0, chunk), :]
            ridx = ridx_ref[pl.ds(row0, chunk), :]
            pc = probs_ref[pl.ds(row0, chunk), :]
            col = lax.broadcasted_iota(jnp.int32, (chunk, n_experts), 1)
            acc = jnp.zeros((chunk, d_ff), jnp.float32)
            for le in range(n_loc):
                e = my * n_loc + le
                prob_e = jnp.sum(
                    jnp.where(col == e, pc, 0.0), axis=1, keepdims=True
                )
                gate = jnp.where(ridx == e, prob_e, 0.0)
                xs = (xc * gate).astype(jnp.bfloat16)
                acc = acc + jnp.dot(
                    xs, ew_bf_ref[le], preferred_element_type=jnp.float32
                )
            return acc

        barrier_sem = pltpu.get_barrier_semaphore()
        for t in range(1, N_DEV):
            pl.semaphore_signal(
                barrier_sem, inc=1,
                device_id=(lax.rem(my + t, N_DEV),),
                device_id_type=pl.DeviceIdType.MESH,
            )
        pl.semaphore_wait(barrier_sem, N_DEV - 1)

        rdmas = []
        for t in range(1, N_DEV):
            dst = lax.rem(my + t, N_DEV)
            acc = partial_chunk(dst)
            if int8_wire:
                amax = jnp.maximum(
                    jnp.max(jnp.abs(acc), axis=1, keepdims=True), 1e-20
                )
                send_ref[t - 1] = jnp.round(acc * (127.0 / amax)).astype(
                    jnp.int8
                )
                sscale_ref[t - 1] = amax * (1.0 / 127.0)
            else:
                send_ref[t - 1] = acc.astype(jnp.bfloat16)
            rdma = pltpu.make_async_remote_copy(
                src_ref=send_ref.at[t - 1],
                dst_ref=recv_ref.at[t - 1],
                send_sem=send_sems.at[t - 1],
                recv_sem=recv_sems.at[t - 1],
                device_id=(dst,),
                device_id_type=pl.DeviceIdType.MESH,
            )
            rdma.start()
            rdmas.append(rdma)
            if int8_wire:
                rdma_s = pltpu.make_async_remote_copy(
                    src_ref=sscale_ref.at[t - 1],
                    dst_ref=rscale_ref.at[t - 1],
                    send_sem=s2_sems.at[t - 1],
                    recv_sem=r2_sems.at[t - 1],
                    device_id=(dst,),
                    device_id_type=pl.DeviceIdType.MESH,
                )
                rdma_s.start()
                rdmas.append(rdma_s)

        own = partial_chunk(my)
        xm = x_ref[pl.ds(my * chunk, chunk), :].astype(jnp.bfloat16)
        own = own + jnp.dot(
            xm, sw_ref[:].astype(jnp.bfloat16),
            preferred_element_type=jnp.float32,
        )

        n_per = 2 if int8_wire else 1
        for t in range(1, N_DEV):
            for r in rdmas[(t - 1) * n_per: t * n_per]:
                r.wait_recv()
            blk = recv_ref[t - 1].astype(jnp.float32)
            if int8_wire:
                blk = blk * rscale_ref[t - 1]
            own = own + blk
        out_ref[:] = own
        for r in rdmas:
            r.wait_send()

    scratch = [
        pltpu.VMEM((n_tok, n_experts), jnp.float32),
        pltpu.VMEM((n_loc, d_model, d_ff), jnp.bfloat16),
        pltpu.VMEM((N_DEV - 1, chunk, d_ff), wire_dt),
        pltpu.VMEM((N_DEV - 1, chunk, d_ff), wire_dt),
        pltpu.VMEM((N_DEV - 1, chunk, 1), jnp.float32),
        pltpu.VMEM((N_DEV - 1, chunk, 1), jnp.float32),
        pltpu.SemaphoreType.DMA((N_DEV - 1,)),
        pltpu.SemaphoreType.DMA((N_DEV - 1,)),
        pltpu.SemaphoreType.DMA((N_DEV - 1,)),
        pltpu.SemaphoreType.DMA((N_DEV - 1,)),
    ]

    return pl.pallas_call(
        body,
        out_shape=jax.ShapeDtypeStruct((chunk, d_ff), jnp.float32),
        in_specs=[pl.BlockSpec(memory_space=pltpu.VMEM)] * 5,
        out_specs=pl.BlockSpec(memory_space=pltpu.VMEM),
        scratch_shapes=scratch,
        compiler_params=pltpu.CompilerParams(collective_id=0),
    )(x, router_W, route_idx, expert_W, shared_W)


# device time: 40159 ns/iter; 1.4876x vs baseline; 1.1086x over previous
import jax
import jax.numpy as jnp
from jax import lax
from jax.experimental import pallas as pl
from jax.experimental.pallas import tpu as pltpu

N_DEV = 4


def kernel(x, router_W, route_idx, expert_W, shared_W):
    n_tok, d_model = x.shape
    n_experts = router_W.shape[1]
    n_loc, _, d_ff = expert_W.shape
    chunk = n_tok // N_DEV

    def body(x_ref, rw_ref, ridx_ref, ew_ref, sw_ref, out_ref,
             coeff_ref, ew_bf_ref, send_ref, recv_ref,
             sscale_ref, rscale_ref,
             send_sems, recv_sems, s2_sems, r2_sems):
        my = lax.axis_index("i")
        e0 = my * n_loc

        scores = jnp.dot(
            x_ref[:].astype(jnp.bfloat16),
            rw_ref[:].astype(jnp.bfloat16),
            preferred_element_type=jnp.float32,
        )
        m = jnp.max(scores, axis=1, keepdims=True)
        p = jnp.exp(scores - m)
        probs = p / jnp.sum(p, axis=1, keepdims=True)
        col = lax.broadcasted_iota(jnp.int32, (n_tok, n_experts), 1)
        mine = (col == ridx_ref[:]) & (col >= e0) & (col < e0 + n_loc)
        coeff_ref[:] = jnp.sum(jnp.where(mine, probs, 0.0), axis=1,
                               keepdims=True)

        for le in range(n_loc):
            ew_bf_ref[pl.ds(le * d_model, d_model), :] = ew_ref[le].astype(
                jnp.bfloat16
            )

        def chunk_y(c):
            row0 = c * chunk
            xcb = x_ref[pl.ds(row0, chunk), :].astype(jnp.bfloat16)
            ridx = ridx_ref[pl.ds(row0, chunk), :]
            zero = jnp.zeros((), jnp.bfloat16)
            xs_cat = jnp.concatenate(
                [jnp.where(ridx == e0 + le, xcb, zero) for le in range(n_loc)],
                axis=1,
            )
            return jnp.dot(xs_cat, ew_bf_ref[:],
                           preferred_element_type=jnp.float32)

        barrier_sem = pltpu.get_barrier_semaphore()
        for t in range(1, N_DEV):
            pl.semaphore_signal(
                barrier_sem, inc=1,
                device_id=(lax.rem(my + t, N_DEV),),
                device_id_type=pl.DeviceIdType.MESH,
            )
        pl.semaphore_wait(barrier_sem, N_DEV - 1)

        rdmas = []
        for t in range(1, N_DEV):
            dst = lax.rem(my + t, N_DEV)
            row0 = dst * chunk
            y = chunk_y(dst)
            ymax = jnp.maximum(
                jnp.max(jnp.abs(y), axis=1, keepdims=True), 1e-20
            )
            send_ref[t - 1] = jnp.round(y * (127.0 / ymax)).astype(jnp.int8)
            sscale_ref[t - 1] = ymax * coeff_ref[pl.ds(row0, chunk), :] * (
                1.0 / 127.0
            )
            rdma = pltpu.make_async_remote_copy(
                src_ref=send_ref.at[t - 1],
                dst_ref=recv_ref.at[t - 1],
                send_sem=send_sems.at[t - 1],
                recv_sem=recv_sems.at[t - 1],
                device_id=(dst,),
                device_id_type=pl.DeviceIdType.MESH,
            )
            rdma.start()
            rdma_s = pltpu.make_async_remote_copy(
                src_ref=sscale_ref.at[t - 1],
                dst_ref=rscale_ref.at[t - 1],
                send_sem=s2_sems.at[t - 1],
                recv_sem=r2_sems.at[t - 1],
                device_id=(dst,),
                device_id_type=pl.DeviceIdType.MESH,
            )
            rdma_s.start()
            rdmas.extend((rdma, rdma_s))

        own = chunk_y(my) * coeff_ref[pl.ds(my * chunk, chunk), :]
        xm = x_ref[pl.ds(my * chunk, chunk), :].astype(jnp.bfloat16)
        own = own + jnp.dot(
            xm, sw_ref[:].astype(jnp.bfloat16),
            preferred_element_type=jnp.float32,
        )

        for t in range(1, N_DEV):
            rdmas[(t - 1) * 2].wait_recv()
            rdmas[(t - 1) * 2 + 1].wait_recv()
            own = own + recv_ref[t - 1].astype(jnp.float32) * rscale_ref[t - 1]
        out_ref[:] = own
        for r in rdmas:
            r.wait_send()

    return pl.pallas_call(
        body,
        out_shape=jax.ShapeDtypeStruct((chunk, d_ff), jnp.float32),
        in_specs=[pl.BlockSpec(memory_space=pltpu.VMEM)] * 5,
        out_specs=pl.BlockSpec(memory_space=pltpu.VMEM),
        scratch_shapes=[
            pltpu.VMEM((n_tok, 1), jnp.float32),
            pltpu.VMEM((n_loc * d_model, d_ff), jnp.bfloat16),
            pltpu.VMEM((N_DEV - 1, chunk, d_ff), jnp.int8),
            pltpu.VMEM((N_DEV - 1, chunk, d_ff), jnp.int8),
            pltpu.VMEM((N_DEV - 1, chunk, 1), jnp.float32),
            pltpu.VMEM((N_DEV - 1, chunk, 1), jnp.float32),
            pltpu.SemaphoreType.DMA((N_DEV - 1,)),
            pltpu.SemaphoreType.DMA((N_DEV - 1,)),
            pltpu.SemaphoreType.DMA((N_DEV - 1,)),
            pltpu.SemaphoreType.DMA((N_DEV - 1,)),
        ],
        compiler_params=pltpu.CompilerParams(collective_id=0),
    )(x, router_W, route_idx, expert_W, shared_W)
